# baseline (speedup 1.0000x reference)
import functools

import jax
import jax.numpy as jnp
from jax import lax
from jax.experimental import pallas as pl
from jax.experimental.pallas import tpu as pltpu
from jax.experimental.pallas import tpu_sc as plsc

_NUM_CORES = 2
_NUM_SUBCORES = 16
_LANES = 16


def _rpe_call(q_len, v_len, max_len, d):
    nw = _NUM_CORES * _NUM_SUBCORES
    assert q_len % nw == 0 and d % _LANES == 0
    rpw = q_len // nw
    p = (max_len - 1) // 2
    off = (q_len - 1) - p
    a_rows = q_len + v_len
    assert a_rows % _NUM_SUBCORES == 0
    bpw = a_rows // _NUM_SUBCORES
    row_w = v_len * d
    nfire = 8

    mesh = plsc.VectorSubcoreMesh(core_axis_name="c", subcore_axis_name="s")

    @functools.partial(
        pl.kernel,
        out_type=jax.ShapeDtypeStruct((q_len, row_w), jnp.float32),
        mesh=mesh,
        compiler_params=pltpu.CompilerParams(use_tc_tiling_on_sc=False),
        scratch_types=[
            pltpu.VMEM((max_len * d,), jnp.float32),
            pltpu.VMEM((bpw * d,), jnp.float32),
            pltpu.VMEM_SHARED((a_rows * d,), jnp.float32),
            pltpu.SemaphoreType.DMA,
            pltpu.SemaphoreType.DMA,
        ],
    )
    def rpe(emb_hbm, out_hbm, emb_v, build_v, a_sh, gsem, csem):
        cid = lax.axis_index("c")
        sid = lax.axis_index("s")
        wid = cid * _NUM_SUBCORES + sid
        i0 = wid * rpw

        pltpu.async_copy(emb_hbm, emb_v, gsem).wait()

        bias = sid * bpw - off

        @pl.loop(0, bpw, step=4)
        def _(t):
            for u in range(4):
                k = jnp.minimum(jnp.maximum(bias + (t + u), 0), max_len - 1)
                for h in range(d // _LANES):
                    build_v[pl.ds((t + u) * d + h * _LANES, _LANES)] = (
                        emb_v[pl.ds(k * d + h * _LANES, _LANES)]
                    )

        pltpu.async_copy(build_v, a_sh.at[pl.ds(sid * bpw * d, bpw * d)], gsem).wait()
        plsc.subcore_barrier()

        def start_of(r):
            return ((q_len - 1) - (i0 + r)) * d

        for b in range(nfire):
            pltpu.async_copy(
                a_sh.at[pl.ds(start_of(b), row_w)], out_hbm.at[i0 + b], csem
            )

        @pl.loop(0, rpw - nfire)
        def _(r):
            pltpu.make_async_copy(
                a_sh.at[pl.ds(start_of(r), row_w)], out_hbm.at[i0 + r], csem
            ).wait()
            pltpu.async_copy(
                a_sh.at[pl.ds(start_of(r + nfire), row_w)],
                out_hbm.at[i0 + r + nfire],
                csem,
            )

        for b in range(nfire):
            r = rpw - nfire + b
            pltpu.make_async_copy(
                a_sh.at[pl.ds(start_of(r), row_w)], out_hbm.at[i0 + r], csem
            ).wait()

    return rpe


def kernel(q, v, embeddings):
    q_len = int(q.shape[1])
    v_len = int(v.shape[1])
    max_len, d = int(embeddings.shape[0]), int(embeddings.shape[1])
    out = _rpe_call(q_len, v_len, max_len, d)(embeddings.reshape(max_len * d))
    return out.reshape(q_len, v_len, d)
